# Initial kernel scaffold; baseline (speedup 1.0000x reference)
import jax
import jax.numpy as jnp
from jax.experimental import pallas as pl


def kernel(norm_adj_matrix, X, W1, W2):
    raise NotImplementedError("write your pallas kernel here")


# speedup vs baseline: 4.1867x; 4.1867x over previous
import jax
import jax.numpy as jnp
from jax.experimental import pallas as pl

_f32 = jnp.float32


def _softmax_kernel(t_ref, sqc_ref, sqr_ref, o_ref):
    d = jnp.maximum((sqc_ref[...] + sqr_ref[...]) - 2.0 * t_ref[...], 0.0)
    m = jnp.min(d, axis=1, keepdims=True)
    e = jnp.exp(m - d)
    s = jnp.sum(e, axis=1, keepdims=True)
    o_ref[...] = e / s + 1e-10


def kernel(norm_adj_matrix, X, W1, W2):
    n = X.shape[0]

    bm = 200 if n % 200 == 0 else n
    mblk = n // bm

    e1 = jax.nn.relu(norm_adj_matrix @ (X @ W1))
    e2 = norm_adj_matrix @ (e1 @ W2)

    sq = jnp.sum(e2 * e2, axis=1)
    t = jax.lax.dot_general(e2, e2, (((1,), (1,)), ((), ())))

    out = pl.pallas_call(
        _softmax_kernel,
        grid=(mblk,),
        in_specs=[
            pl.BlockSpec((bm, n), lambda i: (i, 0)),
            pl.BlockSpec((bm, 1), lambda i: (i, 0)),
            pl.BlockSpec((1, n), lambda i: (0, 0)),
        ],
        out_specs=pl.BlockSpec((bm, n), lambda i: (i, 0)),
        out_shape=jax.ShapeDtypeStruct((n, n), _f32),
    )(t, sq[:, None], sq[None, :])

    return out
